# baseline (speedup 1.0000x reference)
import jax
import jax.numpy as jnp
from jax.experimental import pallas as pl
from jax.experimental.pallas import tpu as pltpu

ALPHA = 0.2
N = 1024
NFEAT = 256
NHID = 64
NHEADS = 3
NCLASS = 40
LOG2E = 1.4426950408889634
NCHUNK = 4
CH = N // NCHUNK

_DOT_T = (((1,), (1,)), ((), ()))


def _elu(x):
    return jnp.where(x >= 0, x, jnp.exp(x) - 1.0)


def _masked_exp(z, maskf):
    slope = jnp.where(z >= 0, -LOG2E, -ALPHA * LOG2E)
    return jnp.exp2(z * slope) * maskf


def _body(x_hbm, m_hbm, W0_h, W1_h, W2_h, a0_h, a1_h, a2_h, Wo_h, ao_h,
          out_ref, xv, mv, Wv, av, Wov, aov, sems):
    cps = [
        pltpu.make_async_copy(x_hbm, xv, sems.at[0]),
        pltpu.make_async_copy(W0_h, Wv.at[0], sems.at[1]),
        pltpu.make_async_copy(W1_h, Wv.at[1], sems.at[2]),
        pltpu.make_async_copy(W2_h, Wv.at[2], sems.at[3]),
        pltpu.make_async_copy(a0_h, av.at[0:1, :], sems.at[4]),
        pltpu.make_async_copy(a1_h, av.at[1:2, :], sems.at[5]),
        pltpu.make_async_copy(a2_h, av.at[2:3, :], sems.at[6]),
        pltpu.make_async_copy(Wo_h, Wov, sems.at[7]),
        pltpu.make_async_copy(ao_h, aov.at[0:1, :], sems.at[8]),
    ]
    mcps = [
        pltpu.make_async_copy(m_hbm.at[c * CH:(c + 1) * CH, :],
                              mv.at[c * CH:(c + 1) * CH, :], sems.at[9 + c])
        for c in range(NCHUNK)
    ]
    for c in cps + mcps:
        c.start()
    for c in cps:
        c.wait()

    x = xv[...]
    ones_col = jnp.ones((N, 1), jnp.float32)
    hps = []
    fs = []
    gs = []
    for k in range(NHEADS):
        h = jax.lax.dot_general(x, Wv[k], _DOT_T,
                                preferred_element_type=jnp.float32)
        ak = av[k, :]
        al = ak[:NHID].reshape(NHID, 1)
        ar = ak[NHID:].reshape(NHID, 1)
        hp = jnp.concatenate([h, ones_col], axis=1)
        fs.append(jnp.dot(h, al, preferred_element_type=jnp.float32))
        gs.append(jnp.dot(h, ar, preferred_element_type=jnp.float32)
                  .reshape(1, N))
        hps.append(hp)

    aggs = [[] for _ in range(NHEADS)]
    for c in range(NCHUNK):
        mcps[c].wait()
        mc = mv[c * CH:(c + 1) * CH, :].astype(jnp.float32)
        for k in range(NHEADS):
            z = fs[k][c * CH:(c + 1) * CH, :] + gs[k]
            S = _masked_exp(z, mc)
            aggs[k].append(jnp.dot(S, hps[k],
                                   preferred_element_type=jnp.float32))
    heads = []
    for k in range(NHEADS):
        agg = jnp.concatenate(aggs[k], axis=0)
        rinv = 1.0 / agg[:, NHID:NHID + 1]
        heads.append(_elu(agg[:, :NHID] * rinv))
    hcat = jnp.concatenate(heads, axis=1)

    maskf = mv[...].astype(jnp.float32)
    ho = jax.lax.dot_general(hcat, Wov[...], _DOT_T,
                             preferred_element_type=jnp.float32)
    ao = aov[0, :]
    fo = jnp.dot(ho, ao[:NCLASS].reshape(NCLASS, 1),
                 preferred_element_type=jnp.float32)
    go = jnp.dot(ho, ao[NCLASS:2 * NCLASS].reshape(NCLASS, 1),
                 preferred_element_type=jnp.float32).reshape(1, N)
    S = _masked_exp(fo + go, maskf)
    hop = jnp.concatenate([ho, ones_col], axis=1)
    agg = jnp.dot(S, hop, preferred_element_type=jnp.float32)
    rinv = 1.0 / agg[:, NCLASS:NCLASS + 1]
    out = _elu(agg[:, :NCLASS] * rinv)
    out_ref[...] = jax.nn.log_softmax(out, axis=1).T


def kernel(x, adj, W0, W1, W2, a0, a1, a2, W_out, a_out):
    res = pl.pallas_call(
        _body,
        in_specs=[pl.BlockSpec(memory_space=pl.ANY)] * 10,
        out_shape=jax.ShapeDtypeStruct((NCLASS, N), jnp.float32),
        scratch_shapes=[
            pltpu.VMEM((N, NFEAT), jnp.float32),
            pltpu.VMEM((N, N), jnp.int32),
            pltpu.VMEM((NHEADS, NHID, NFEAT), jnp.float32),
            pltpu.VMEM((NHEADS, 2 * NHID), jnp.float32),
            pltpu.VMEM((NCLASS, NHID * NHEADS), jnp.float32),
            pltpu.VMEM((1, 2 * NCLASS), jnp.float32),
            pltpu.SemaphoreType.DMA((9 + NCHUNK,)),
        ],
    )(*[pltpu.with_memory_space_constraint(v, pltpu.MemorySpace.HBM)
        for v in (x, adj.astype(jnp.int32), W0.T, W1.T, W2.T, a0, a1, a2,
                  W_out.T, a_out)])
    return res.T


# speedup vs baseline: 1.0824x; 1.0824x over previous
import jax
import jax.numpy as jnp
from jax.experimental import pallas as pl
from jax.experimental.pallas import tpu as pltpu

ALPHA = 0.2
N = 1024
NFEAT = 256
NHID = 64
NHEADS = 3
NCLASS = 40
LOG2E = 1.4426950408889634
NCHUNK = 4
CH = N // NCHUNK

_DOT_T = (((1,), (1,)), ((), ()))


def _elu(x):
    return jnp.where(x >= 0, x, jnp.exp(x) - 1.0)


def _masked_exp(z, maskf):
    slope = jnp.where(z >= 0, jnp.bfloat16(-LOG2E),
                      jnp.bfloat16(-ALPHA * LOG2E))
    return jnp.exp2(z * slope) * maskf


def _body(x_hbm, m_hbm, W0_h, W1_h, W2_h, a0_h, a1_h, a2_h, Wo_h, ao_h,
          out_ref, xv, mv, Wv, av, Wov, aov, sems):
    cps = [
        pltpu.make_async_copy(x_hbm, xv, sems.at[0]),
        pltpu.make_async_copy(W0_h, Wv.at[0], sems.at[1]),
        pltpu.make_async_copy(W1_h, Wv.at[1], sems.at[2]),
        pltpu.make_async_copy(W2_h, Wv.at[2], sems.at[3]),
        pltpu.make_async_copy(a0_h, av.at[0:1, :], sems.at[4]),
        pltpu.make_async_copy(a1_h, av.at[1:2, :], sems.at[5]),
        pltpu.make_async_copy(a2_h, av.at[2:3, :], sems.at[6]),
        pltpu.make_async_copy(Wo_h, Wov, sems.at[7]),
        pltpu.make_async_copy(ao_h, aov.at[0:1, :], sems.at[8]),
    ]
    mcps = [
        pltpu.make_async_copy(m_hbm.at[c * CH:(c + 1) * CH, :],
                              mv.at[c * CH:(c + 1) * CH, :], sems.at[9 + c])
        for c in range(NCHUNK)
    ]
    for c in cps + mcps:
        c.start()
    for c in cps:
        c.wait()

    x = xv[...]
    ones_col = jnp.ones((N, 1), jnp.float32)
    hps = []
    fs = []
    gs = []
    for k in range(NHEADS):
        h = jax.lax.dot_general(x, Wv[k], _DOT_T,
                                preferred_element_type=jnp.float32)
        ak = av[k, :]
        al = ak[:NHID].reshape(NHID, 1)
        ar = ak[NHID:].reshape(NHID, 1)
        hp = jnp.concatenate([h, ones_col], axis=1)
        fs.append(jnp.dot(h, al, preferred_element_type=jnp.float32)
                  .astype(jnp.bfloat16))
        gs.append(jnp.dot(h, ar, preferred_element_type=jnp.float32)
                  .reshape(1, N).astype(jnp.bfloat16))
        hps.append(hp.astype(jnp.bfloat16))

    aggs = [[] for _ in range(NHEADS)]
    for c in range(NCHUNK):
        mcps[c].wait()
        mc = mv[c * CH:(c + 1) * CH, :].astype(jnp.bfloat16)
        for k in range(NHEADS):
            z = fs[k][c * CH:(c + 1) * CH, :] + gs[k]
            S = _masked_exp(z, mc)
            aggs[k].append(jnp.dot(S, hps[k],
                                   preferred_element_type=jnp.float32))
    heads = []
    for k in range(NHEADS):
        agg = jnp.concatenate(aggs[k], axis=0)
        rinv = 1.0 / agg[:, NHID:NHID + 1]
        heads.append(_elu(agg[:, :NHID] * rinv))
    hcat = jnp.concatenate(heads, axis=1)

    maskf = mv[...].astype(jnp.bfloat16)
    ho = jax.lax.dot_general(hcat, Wov[...], _DOT_T,
                             preferred_element_type=jnp.float32)
    ao = aov[0, :]
    fo = jnp.dot(ho, ao[:NCLASS].reshape(NCLASS, 1),
                 preferred_element_type=jnp.float32).astype(jnp.bfloat16)
    go = jnp.dot(ho, ao[NCLASS:2 * NCLASS].reshape(NCLASS, 1),
                 preferred_element_type=jnp.float32
                 ).reshape(1, N).astype(jnp.bfloat16)
    S = _masked_exp(fo + go, maskf)
    hop = jnp.concatenate([ho, ones_col], axis=1).astype(jnp.bfloat16)
    agg = jnp.dot(S, hop, preferred_element_type=jnp.float32)
    rinv = 1.0 / agg[:, NCLASS:NCLASS + 1]
    out = _elu(agg[:, :NCLASS] * rinv)
    out_ref[...] = jax.nn.log_softmax(out, axis=1).T


def kernel(x, adj, W0, W1, W2, a0, a1, a2, W_out, a_out):
    res = pl.pallas_call(
        _body,
        in_specs=[pl.BlockSpec(memory_space=pl.ANY)] * 10,
        out_shape=jax.ShapeDtypeStruct((NCLASS, N), jnp.float32),
        scratch_shapes=[
            pltpu.VMEM((N, NFEAT), jnp.float32),
            pltpu.VMEM((N, N), jnp.int32),
            pltpu.VMEM((NHEADS, NHID, NFEAT), jnp.float32),
            pltpu.VMEM((NHEADS, 2 * NHID), jnp.float32),
            pltpu.VMEM((NCLASS, NHID * NHEADS), jnp.float32),
            pltpu.VMEM((1, 2 * NCLASS), jnp.float32),
            pltpu.SemaphoreType.DMA((9 + NCHUNK,)),
        ],
    )(*[pltpu.with_memory_space_constraint(v, pltpu.MemorySpace.HBM)
        for v in (x, adj.astype(jnp.int32), W0.T, W1.T, W2.T, a0, a1, a2,
                  W_out.T, a_out)])
    return res.T
